# baseline (speedup 1.0000x reference)
import jax
import jax.numpy as jnp
from jax.experimental import pallas as pl

_CENTER_VAR = 0.1
_SIZE_VAR = 0.2


def _body(conf_ref, loc_ref, pri_ref, scores_ref, boxes_ref):
    x = conf_ref[...]
    m = jnp.max(x, axis=-1, keepdims=True)
    e = jnp.exp(x - m)
    s = jnp.sum(e, axis=-1, keepdims=True)
    scores_ref[...] = e * (1.0 / s)

    loc = loc_ref[...]
    pr = pri_ref[...]
    xy = loc[:, :2] * _CENTER_VAR * pr[:, 2:] + pr[:, :2]
    half = jnp.exp(loc[:, 2:] * _SIZE_VAR) * pr[:, 2:] * 0.5
    boxes_ref[...] = jnp.concatenate([xy - half, xy + half], axis=-1)


def kernel(location_preds, confidence_preds, priors):
    B, N, C = confidence_preds.shape
    conf2 = confidence_preds.reshape(B * N, C)
    loc2 = location_preds.reshape(B * N, 4)
    R = 5000
    n_per = N // R
    grid = (B * N) // R
    scores, boxes = pl.pallas_call(
        _body,
        grid=(grid,),
        in_specs=[
            pl.BlockSpec((R, C), lambda i: (i, 0)),
            pl.BlockSpec((R, 4), lambda i: (i, 0)),
            pl.BlockSpec((R, 4), lambda i: (i % n_per, 0)),
        ],
        out_specs=[
            pl.BlockSpec((R, C), lambda i: (i, 0)),
            pl.BlockSpec((R, 4), lambda i: (i, 0)),
        ],
        out_shape=[
            jax.ShapeDtypeStruct((B * N, C), jnp.float32),
            jax.ShapeDtypeStruct((B * N, 4), jnp.float32),
        ],
    )(conf2, loc2, priors)
    return scores.reshape(B, N, C), boxes.reshape(B, N, 4)


# speedup vs baseline: 1.0154x; 1.0154x over previous
import jax
import jax.numpy as jnp
from jax.experimental import pallas as pl

_CENTER_VAR = 0.1
_SIZE_VAR = 0.2


def _body(conf_ref, loc_ref, pri_ref, scores_ref, boxes_ref):
    x = conf_ref[...]
    C = x.shape[-1]
    e = jnp.exp(x)
    ones = jnp.ones((C, C), dtype=jnp.float32)
    s = jax.lax.dot_general(e, ones, (((1,), (0,)), ((), ())),
                            preferred_element_type=jnp.float32)
    scores_ref[...] = e * (1.0 / s)

    loc = loc_ref[...]
    pr = pri_ref[...]
    xy = loc[:, :2] * _CENTER_VAR * pr[:, 2:] + pr[:, :2]
    half = jnp.exp(loc[:, 2:] * _SIZE_VAR) * pr[:, 2:] * 0.5
    boxes_ref[...] = jnp.concatenate([xy - half, xy + half], axis=-1)


def kernel(location_preds, confidence_preds, priors):
    B, N, C = confidence_preds.shape
    conf2 = confidence_preds.reshape(B * N, C)
    loc2 = location_preds.reshape(B * N, 4)
    R = 5000
    n_per = N // R
    grid = (B * N) // R
    scores, boxes = pl.pallas_call(
        _body,
        grid=(grid,),
        in_specs=[
            pl.BlockSpec((R, C), lambda i: (i, 0)),
            pl.BlockSpec((R, 4), lambda i: (i, 0)),
            pl.BlockSpec((R, 4), lambda i: (i % n_per, 0)),
        ],
        out_specs=[
            pl.BlockSpec((R, C), lambda i: (i, 0)),
            pl.BlockSpec((R, 4), lambda i: (i, 0)),
        ],
        out_shape=[
            jax.ShapeDtypeStruct((B * N, C), jnp.float32),
            jax.ShapeDtypeStruct((B * N, 4), jnp.float32),
        ],
    )(conf2, loc2, priors)
    return scores.reshape(B, N, C), boxes.reshape(B, N, 4)


# speedup vs baseline: 12.2972x; 12.1107x over previous
import jax
import jax.numpy as jnp
from jax.experimental import pallas as pl

_CENTER_VAR = 0.1
_SIZE_VAR = 0.2
_NB = 1024


def _body(conf_ref, loc_ref, pri_ref, scores_ref, boxes_ref):
    x = conf_ref[...]
    e = jnp.exp(x)
    s = jnp.sum(e, axis=0, keepdims=True)
    scores_ref[...] = e * (1.0 / s)

    loc = loc_ref[...]
    pr = pri_ref[...]
    lx, ly = loc[:, 0, :], loc[:, 1, :]
    lw, lh = loc[:, 2, :], loc[:, 3, :]
    px, py, pw, ph = pr[0], pr[1], pr[2], pr[3]
    cx = lx * _CENTER_VAR * pw[None] + px[None]
    cy = ly * _CENTER_VAR * ph[None] + py[None]
    hw = jnp.exp(lw * _SIZE_VAR) * pw[None] * 0.5
    hh = jnp.exp(lh * _SIZE_VAR) * ph[None] * 0.5
    boxes_ref[...] = jnp.concatenate(
        [(cx - hw)[:, None, :], (cy - hh)[:, None, :],
         (cx + hw)[:, None, :], (cy + hh)[:, None, :]], axis=1)


def kernel(location_preds, confidence_preds, priors):
    B, N, C = confidence_preds.shape
    conf_t = jnp.transpose(confidence_preds, (2, 0, 1))
    loc_t = jnp.transpose(location_preds, (0, 2, 1))
    pri_t = jnp.transpose(priors, (1, 0))
    grid = pl.cdiv(N, _NB)
    scores_t, boxes_t = pl.pallas_call(
        _body,
        grid=(grid,),
        in_specs=[
            pl.BlockSpec((C, B, _NB), lambda i: (0, 0, i)),
            pl.BlockSpec((B, 4, _NB), lambda i: (0, 0, i)),
            pl.BlockSpec((4, _NB), lambda i: (0, i)),
        ],
        out_specs=[
            pl.BlockSpec((C, B, _NB), lambda i: (0, 0, i)),
            pl.BlockSpec((B, 4, _NB), lambda i: (0, 0, i)),
        ],
        out_shape=[
            jax.ShapeDtypeStruct((C, B, N), jnp.float32),
            jax.ShapeDtypeStruct((B, 4, N), jnp.float32),
        ],
    )(conf_t, loc_t, pri_t)
    scores = jnp.transpose(scores_t, (1, 2, 0))
    boxes = jnp.transpose(boxes_t, (0, 2, 1))
    return scores, boxes


# speedup vs baseline: 12.9601x; 1.0539x over previous
import jax
import jax.numpy as jnp
from jax.experimental import pallas as pl

_CENTER_VAR = 0.1
_SIZE_VAR = 0.2
_NB = 2048


def _body(conf_ref, loc_ref, pri_ref, scores_ref, boxes_ref):
    x = conf_ref[...]
    e = jnp.exp(x)
    s = jnp.sum(e, axis=0, keepdims=True)
    scores_ref[...] = e * (1.0 / s)

    loc = loc_ref[...]
    pr = pri_ref[...]
    lx, ly = loc[:, 0, :], loc[:, 1, :]
    lw, lh = loc[:, 2, :], loc[:, 3, :]
    px, py, pw, ph = pr[0], pr[1], pr[2], pr[3]
    cx = lx * _CENTER_VAR * pw[None] + px[None]
    cy = ly * _CENTER_VAR * ph[None] + py[None]
    hw = jnp.exp(lw * _SIZE_VAR) * pw[None] * 0.5
    hh = jnp.exp(lh * _SIZE_VAR) * ph[None] * 0.5
    boxes_ref[...] = jnp.concatenate(
        [(cx - hw)[:, None, :], (cy - hh)[:, None, :],
         (cx + hw)[:, None, :], (cy + hh)[:, None, :]], axis=1)


def kernel(location_preds, confidence_preds, priors):
    B, N, C = confidence_preds.shape
    conf_t = jnp.transpose(confidence_preds, (2, 0, 1))
    loc_t = jnp.transpose(location_preds, (0, 2, 1))
    pri_t = jnp.transpose(priors, (1, 0))
    grid = pl.cdiv(N, _NB)
    scores_t, boxes_t = pl.pallas_call(
        _body,
        grid=(grid,),
        in_specs=[
            pl.BlockSpec((C, B, _NB), lambda i: (0, 0, i)),
            pl.BlockSpec((B, 4, _NB), lambda i: (0, 0, i)),
            pl.BlockSpec((4, _NB), lambda i: (0, i)),
        ],
        out_specs=[
            pl.BlockSpec((C, B, _NB), lambda i: (0, 0, i)),
            pl.BlockSpec((B, 4, _NB), lambda i: (0, 0, i)),
        ],
        out_shape=[
            jax.ShapeDtypeStruct((C, B, N), jnp.float32),
            jax.ShapeDtypeStruct((B, 4, N), jnp.float32),
        ],
    )(conf_t, loc_t, pri_t)
    scores = jnp.transpose(scores_t, (1, 2, 0))
    boxes = jnp.transpose(boxes_t, (0, 2, 1))
    return scores, boxes
